# baseline (speedup 1.0000x reference)
import functools

import jax
import jax.numpy as jnp
from jax import lax
from jax.experimental import pallas as pl
from jax.experimental.pallas import tpu as pltpu
from jax.experimental.pallas import tpu_sc as plsc

B = 4096
D_USER, D_ITEM, D_CAT = 64, 64, 32
D_OUT = D_USER + D_ITEM + D_CAT

NC, NS = 2, 16
NW = NC * NS
B_PER_W = B // NW


@functools.lru_cache(maxsize=1)
def _build():
    mesh = plsc.VectorSubcoreMesh(core_axis_name="c", subcore_axis_name="s")

    @functools.partial(
        pl.kernel,
        mesh=mesh,
        compiler_params=pltpu.CompilerParams(use_tc_tiling_on_sc=False),
        out_type=jax.ShapeDtypeStruct((B, D_OUT), jnp.float32),
        scratch_types=[
            pltpu.VMEM((B_PER_W,), jnp.int32),
            pltpu.VMEM((B_PER_W,), jnp.int32),
            pltpu.VMEM((B_PER_W,), jnp.int32),
            pltpu.VMEM((B_PER_W, D_USER), jnp.float32),
            pltpu.VMEM((B_PER_W, D_ITEM), jnp.float32),
            pltpu.VMEM((B_PER_W, D_CAT), jnp.float32),
            pltpu.SemaphoreType.DMA,
        ],
    )
    def k(uid_hbm, iid_hbm, cid_hbm, wu_hbm, wi_hbm, wc_hbm, out_hbm,
          idx_u, idx_i, idx_c, u_v, i_v, c_v, sem):
        wid = lax.axis_index("s") * NC + lax.axis_index("c")
        base = wid * B_PER_W
        pltpu.sync_copy(uid_hbm.at[pl.ds(base, B_PER_W)], idx_u)
        pltpu.sync_copy(iid_hbm.at[pl.ds(base, B_PER_W)], idx_i)
        pltpu.sync_copy(cid_hbm.at[pl.ds(base, B_PER_W)], idx_c)
        cu = pltpu.async_copy(wu_hbm.at[idx_u], u_v, sem)
        ci = pltpu.async_copy(wi_hbm.at[idx_i], i_v, sem)
        cc = pltpu.async_copy(wc_hbm.at[idx_c], c_v, sem)
        cu.wait()
        ci.wait()
        cc.wait()
        pltpu.sync_copy(u_v, out_hbm.at[pl.ds(base, B_PER_W), pl.ds(0, D_USER)])
        pltpu.sync_copy(i_v, out_hbm.at[pl.ds(base, B_PER_W), pl.ds(D_USER, D_ITEM)])
        pltpu.sync_copy(c_v, out_hbm.at[pl.ds(base, B_PER_W), pl.ds(D_USER + D_ITEM, D_CAT)])

    return k


def kernel(user_id, item_id, category, W_user, W_item, W_cat):
    k = _build()
    return k(user_id.astype(jnp.int32), item_id.astype(jnp.int32),
             category.astype(jnp.int32), W_user, W_item, W_cat)


# speedup vs baseline: 2.2040x; 2.2040x over previous
import functools

import jax
import jax.numpy as jnp
from jax import lax
from jax.experimental import pallas as pl
from jax.experimental.pallas import tpu as pltpu
from jax.experimental.pallas import tpu_sc as plsc

B = 4096
D_USER, D_ITEM, D_CAT = 64, 64, 32

NC, NS = 2, 16
NW = NC * NS
B_PER_W = B // NW
CHUNK = 32
N_CHUNKS = B_PER_W // CHUNK
L = 16


@functools.lru_cache(maxsize=1)
def _build():
    mesh = plsc.VectorSubcoreMesh(core_axis_name="c", subcore_axis_name="s")

    @functools.partial(
        pl.kernel,
        mesh=mesh,
        out_type=(
            jax.ShapeDtypeStruct((B, D_USER), jnp.float32),
            jax.ShapeDtypeStruct((B, D_ITEM), jnp.float32),
            jax.ShapeDtypeStruct((B, D_CAT), jnp.float32),
        ),
        scratch_types=[
            pltpu.VMEM((B_PER_W,), jnp.int32),
            pltpu.VMEM((CHUNK, 8, D_USER), jnp.float32),
            pltpu.VMEM((CHUNK, 8, D_CAT), jnp.float32),
            pltpu.VMEM((B_PER_W, D_USER), jnp.float32),
            pltpu.VMEM((B_PER_W, D_ITEM), jnp.float32),
            pltpu.VMEM((B_PER_W, D_CAT), jnp.float32),
            pltpu.SemaphoreType.DMA,
        ],
    )
    def k(uid_hbm, iid_hbm, cid_hbm, wu_hbm, wi_hbm, wc_hbm,
          ou_hbm, oi_hbm, oc_hbm,
          idx_v, big_blk, cat_blk, u_v, i_v, c_v, sem):
        wid = lax.axis_index("s") * NC + lax.axis_index("c")
        base = wid * B_PER_W

        def field(id_hbm, tbl_hbm, out_v, blk_buf, d):
            pltpu.sync_copy(id_hbm.at[pl.ds(base, B_PER_W)], idx_v)

            def chunk_body(c, _):
                def issue(g, _):
                    ids = idx_v[pl.ds(c * CHUNK + g * L, L)]
                    blks = lax.shift_right_logical(ids, 3)
                    for j in range(L):
                        pltpu.async_copy(tbl_hbm.at[blks[j]],
                                         blk_buf.at[g * L + j], sem)
                    return 0

                lax.fori_loop(0, CHUNK // L, issue, 0)

                def drain(j, _):
                    pltpu.make_async_copy(tbl_hbm.at[0], blk_buf.at[j], sem).wait()
                    return 0

                lax.fori_loop(0, CHUNK, drain, 0)

                def select(g, _):
                    ids = idx_v[pl.ds(c * CHUNK + g * L, L)]
                    rs = lax.rem(ids, 8)
                    for j in range(L):
                        cj = c * CHUNK + g * L + j
                        r = rs[j]
                        for q in range(d // L):
                            out_v[cj, pl.ds(q * L, L)] = (
                                blk_buf[g * L + j, r, pl.ds(q * L, L)])
                    return 0

                lax.fori_loop(0, CHUNK // L, select, 0)
                return 0

            lax.fori_loop(0, N_CHUNKS, chunk_body, 0)

        field(uid_hbm, wu_hbm, u_v, big_blk, D_USER)
        pltpu.sync_copy(u_v, ou_hbm.at[pl.ds(base, B_PER_W)])
        field(iid_hbm, wi_hbm, i_v, big_blk, D_ITEM)
        pltpu.sync_copy(i_v, oi_hbm.at[pl.ds(base, B_PER_W)])
        field(cid_hbm, wc_hbm, c_v, cat_blk, D_CAT)
        pltpu.sync_copy(c_v, oc_hbm.at[pl.ds(base, B_PER_W)])

    return k


def kernel(user_id, item_id, category, W_user, W_item, W_cat):
    k = _build()
    wu3 = W_user.reshape(W_user.shape[0] // 8, 8, D_USER)
    wi3 = W_item.reshape(W_item.shape[0] // 8, 8, D_ITEM)
    wc3 = W_cat.reshape(W_cat.shape[0] // 8, 8, D_CAT)
    ou, oi, oc = k(user_id.astype(jnp.int32), item_id.astype(jnp.int32),
                   category.astype(jnp.int32), wu3, wi3, wc3)
    return jnp.concatenate([ou, oi, oc], axis=-1)
